# baseline (speedup 1.0000x reference)
import math

import jax
import jax.numpy as jnp
import numpy as np
from jax import lax
from jax.experimental import pallas as pl
from jax.experimental.pallas import tpu as pltpu
from jax.experimental.pallas import tpu_sc as plsc

_FIELD_DIMS = [3846] * 25 + [3850]
_V = sum(_FIELD_DIMS)
_F = 26
_D = 16
_B = 4096
_OFFS = np.concatenate([[0], np.cumsum(_FIELD_DIMS)[:-1]]).astype(np.int32)

_NC = 2
_NS = 16
_NW = _NC * _NS
_RPW = _B // _NW
_C = 4
_CH = _RPW // _C
_IDXN = _C * _F
_V8 = _V // 8


_VP = _V // 8


def _fire(table, fctab, idx_v, slab, fcbuf, sem):
    descs = []
    for i in range(_F):
        descs.append(
            pltpu.make_async_copy(
                table.at[i].at[idx_v], slab.at[pl.ds(i * _IDXN, _IDXN)], sem
            )
        )
    descs.append(pltpu.make_async_copy(fctab.at[idx_v], fcbuf, sem))
    return descs


_PAIRS = [(a, a + delta) for delta in range(1, _F) for a in range(_F - delta)]


def _sc_body(table, fctab, xo, s_out, r_out,
             idx0, idx1, slab0, slab1, fc0, fc1, sbuf, rbuf, sem0, sem1):
    wid = lax.axis_index("s") * _NC + lax.axis_index("c")
    row0 = wid * _RPW
    idx_b = [idx0, idx1]
    slab_b = [slab0, slab1]
    fc_b = [fc0, fc1]
    sem_b = [sem0, sem1]

    pltpu.sync_copy(xo.at[pl.ds(row0 * _F, _IDXN)], idx0)
    for d in _fire(table, fctab, idx0, slab0, fc0, sem0):
        d.start()

    def phase(g, p):
        q = 1 - p

        def stage():
            pltpu.sync_copy(
                xo.at[pl.ds((row0 + (g + 1) * _C) * _F, _IDXN)], idx_b[q]
            )
            for d in _fire(table, fctab, idx_b[q], slab_b[q], fc_b[q], sem_b[q]):
                d.start()

        if p == 0:
            stage()
        else:
            @pl.when(g + 1 < _CH)
            def _():
                stage()

        for d in _fire(table, fctab, idx_b[p], slab_b[p], fc_b[p], sem_b[p]):
            d.wait()

        slab = slab_b[p]
        fcbuf = fc_b[p]

        def row_body(c, carry2):
            base = c * _F
            for j in range(_F):
                sbuf[c, pl.ds(j * 16, 16)] = slab[j * _IDXN + base + j, :]
            ffm = jnp.zeros((16,), jnp.float32)
            lin = jnp.zeros((16,), jnp.float32)
            for (ii, jj) in _PAIRS:
                u = slab[ii * _IDXN + base + jj, :]
                v = slab[jj * _IDXN + base + ii, :]
                ffm = ffm + u * v
                plsc.addupdate(sbuf.at[c, pl.ds(jj * 16, 16)], u)
                plsc.addupdate(sbuf.at[c, pl.ds(ii * 16, 16)], v)
            for j in range(_F):
                lin = lin + fcbuf[base + j, :]
            rbuf[c, :] = ffm + lin * (1.0 / 16.0)
            return carry2

        lax.fori_loop(0, _C, row_body, 0)
        pltpu.sync_copy(sbuf, s_out.at[pl.ds(row0 + g * _C, _C)])
        pltpu.sync_copy(rbuf, r_out.at[pl.ds(row0 + g * _C, _C)])

    def chunk2_body(gg, carry):
        phase(2 * gg, 0)
        phase(2 * gg + 1, 1)
        return carry

    lax.fori_loop(0, _CH // 2, chunk2_body, 0)


_sc_call = pl.kernel(
    _sc_body,
    out_type=[
        jax.ShapeDtypeStruct((_B, _F * _D), jnp.float32),
        jax.ShapeDtypeStruct((_B, 16), jnp.float32),
    ],
    mesh=plsc.VectorSubcoreMesh(
        core_axis_name="c", subcore_axis_name="s", num_cores=_NC, num_subcores=_NS
    ),
    scratch_types=[
        pltpu.VMEM((_IDXN,), jnp.int32),
        pltpu.VMEM((_IDXN,), jnp.int32),
        pltpu.VMEM((_F * _IDXN, _D), jnp.float32),
        pltpu.VMEM((_F * _IDXN, _D), jnp.float32),
        pltpu.VMEM((_IDXN, 16), jnp.float32),
        pltpu.VMEM((_IDXN, 16), jnp.float32),
        pltpu.VMEM((_C, _F * _D), jnp.float32),
        pltpu.VMEM((_C, 16), jnp.float32),
        pltpu.SemaphoreType.DMA,
        pltpu.SemaphoreType.DMA,
    ],
    compiler_params=pltpu.CompilerParams(use_tc_tiling_on_sc=False),
)


def _mlp_body(s_ref, r_ref, w1, b1, w2, b2, w3, b3, wout, o_ref):
    h = jnp.dot(s_ref[...], w1[...], preferred_element_type=jnp.float32) + b1[...]
    h = jnp.maximum(h, 0.0)
    h = jnp.dot(h, w2[...], preferred_element_type=jnp.float32) + b2[...]
    h = jnp.maximum(h, 0.0)
    h = jnp.dot(h, w3[...], preferred_element_type=jnp.float32) + b3[...]
    h = jnp.maximum(h, 0.0)
    o = jnp.dot(h, wout[...], preferred_element_type=jnp.float32)
    o_ref[...] = o[:, 0] + jnp.sum(r_ref[...], axis=1)


_BLK = 512


def _mlp_call(s, r, w1, b1, w2, b2, w3, b3, wout):
    full = lambda i: (0, 0)
    return pl.pallas_call(
        _mlp_body,
        grid=(_B // _BLK,),
        in_specs=[
            pl.BlockSpec((_BLK, _F * _D), lambda i: (i, 0)),
            pl.BlockSpec((_BLK, 16), lambda i: (i, 0)),
            pl.BlockSpec((_F * _D, 400), full),
            pl.BlockSpec((400,), lambda i: (0,)),
            pl.BlockSpec((400, 400), full),
            pl.BlockSpec((400,), lambda i: (0,)),
            pl.BlockSpec((400, 400), full),
            pl.BlockSpec((400,), lambda i: (0,)),
            pl.BlockSpec((400, 1), full),
        ],
        out_specs=pl.BlockSpec((_BLK,), lambda i: (i,)),
        out_shape=jax.ShapeDtypeStruct((_B,), jnp.float32),
    )(s, r, w1, b1, w2, b2, w3, b3, wout)


def kernel(x, fc_w, bias, emb, W1, b1, g1, be1, W2, b2, g2, be2, W3, b3, g3, be3, Wout, bout):
    offs = jnp.asarray(_OFFS)
    xo = x + offs[None, :]
    xo_flat = xo.reshape(-1)
    fctab = jnp.broadcast_to(fc_w, (_V, _D))

    s, r = _sc_call(emb, fctab, xo_flat)

    inv = 1.0 / math.sqrt(1.0 + 1e-5)
    s1 = g1 * inv
    s2 = g2 * inv
    s3 = g3 * inv
    w1 = W1 * s1[None, :]
    b1f = b1 * s1 + be1
    w2 = W2 * s2[None, :]
    b2f = b2 * s2 + be2
    w3 = W3 * s3[None, :]
    b3f = b3 * s3 + be3

    out = _mlp_call(s, r, w1, b1f, w2, b2f, w3, b3f, Wout)
    return out + (bias[0] + bout[0])


# speedup vs baseline: 1.1165x; 1.1165x over previous
import math

import jax
import jax.numpy as jnp
import numpy as np
from jax import lax
from jax.experimental import pallas as pl
from jax.experimental.pallas import tpu as pltpu
from jax.experimental.pallas import tpu_sc as plsc

_FIELD_DIMS = [3846] * 25 + [3850]
_V = sum(_FIELD_DIMS)
_F = 26
_D = 16
_B = 4096
_OFFS = np.concatenate([[0], np.cumsum(_FIELD_DIMS)[:-1]]).astype(np.int32)

_NC = 2
_NS = 16
_NW = _NC * _NS
_RPW = _B // _NW
_C = 4
_CH = _RPW // _C
_IDXN = _C * _F
_V8 = _V // 8


_VP = _V // 8


def _fire(table, fctab, idx_v, slab, fcbuf, sem):
    descs = []
    for i in range(_F):
        descs.append(
            pltpu.make_async_copy(
                table.at[i].at[idx_v], slab.at[pl.ds(i * _IDXN, _IDXN)], sem
            )
        )
    descs.append(pltpu.make_async_copy(fctab.at[idx_v], fcbuf, sem))
    return descs


_PAIRS = [(a, a + delta) for delta in range(1, _F) for a in range(_F - delta)]


def _sc_body(table, fctab, xo, s_out, r_out,
             idx0, idx1, slab0, slab1, fc0, fc1, sbuf, rbuf, sem0, sem1):
    wid = lax.axis_index("s") * _NC + lax.axis_index("c")
    row0 = wid * _RPW
    idx_b = [idx0, idx1]
    slab_b = [slab0, slab1]
    fc_b = [fc0, fc1]
    sem_b = [sem0, sem1]

    pltpu.sync_copy(xo.at[pl.ds(row0 * _F, _IDXN)], idx0)
    for d in _fire(table, fctab, idx0, slab0, fc0, sem0):
        d.start()

    def phase(g, p):
        q = 1 - p

        def stage():
            pltpu.sync_copy(
                xo.at[pl.ds((row0 + (g + 1) * _C) * _F, _IDXN)], idx_b[q]
            )
            for d in _fire(table, fctab, idx_b[q], slab_b[q], fc_b[q], sem_b[q]):
                d.start()

        if p == 0:
            stage()
        else:
            @pl.when(g + 1 < _CH)
            def _():
                stage()

        for d in _fire(table, fctab, idx_b[p], slab_b[p], fc_b[p], sem_b[p]):
            d.wait()

        slab = slab_b[p]
        fcbuf = fc_b[p]

        def row_body(c, carry2):
            base = c * _F
            s_cols = [slab[j * _IDXN + base + j, :] for j in range(_F)]
            ffm = jnp.zeros((16,), jnp.float32)
            lin = jnp.zeros((16,), jnp.float32)
            for (ii, jj) in _PAIRS:
                u = slab[ii * _IDXN + base + jj, :]
                v = slab[jj * _IDXN + base + ii, :]
                ffm = ffm + u * v
                s_cols[jj] = s_cols[jj] + u
                s_cols[ii] = s_cols[ii] + v
            for j in range(_F):
                lin = lin + fcbuf[base + j, :]
            rbuf[c, :] = ffm + lin * (1.0 / 16.0)
            for j in range(_F):
                sbuf[c, pl.ds(j * 16, 16)] = s_cols[j]
            return carry2

        lax.fori_loop(0, _C, row_body, 0)
        pltpu.sync_copy(sbuf, s_out.at[pl.ds(row0 + g * _C, _C)])
        pltpu.sync_copy(rbuf, r_out.at[pl.ds(row0 + g * _C, _C)])

    def chunk2_body(gg, carry):
        phase(2 * gg, 0)
        phase(2 * gg + 1, 1)
        return carry

    lax.fori_loop(0, _CH // 2, chunk2_body, 0)


_sc_call = pl.kernel(
    _sc_body,
    out_type=[
        jax.ShapeDtypeStruct((_B, _F * _D), jnp.float32),
        jax.ShapeDtypeStruct((_B, 16), jnp.float32),
    ],
    mesh=plsc.VectorSubcoreMesh(
        core_axis_name="c", subcore_axis_name="s", num_cores=_NC, num_subcores=_NS
    ),
    scratch_types=[
        pltpu.VMEM((_IDXN,), jnp.int32),
        pltpu.VMEM((_IDXN,), jnp.int32),
        pltpu.VMEM((_F * _IDXN, _D), jnp.float32),
        pltpu.VMEM((_F * _IDXN, _D), jnp.float32),
        pltpu.VMEM((_IDXN, 16), jnp.float32),
        pltpu.VMEM((_IDXN, 16), jnp.float32),
        pltpu.VMEM((_C, _F * _D), jnp.float32),
        pltpu.VMEM((_C, 16), jnp.float32),
        pltpu.SemaphoreType.DMA,
        pltpu.SemaphoreType.DMA,
    ],
    compiler_params=pltpu.CompilerParams(use_tc_tiling_on_sc=False),
)


def _mlp_body(s_ref, r_ref, w1, b1, w2, b2, w3, b3, wout, o_ref):
    h = jnp.dot(s_ref[...], w1[...], preferred_element_type=jnp.float32) + b1[...]
    h = jnp.maximum(h, 0.0)
    h = jnp.dot(h, w2[...], preferred_element_type=jnp.float32) + b2[...]
    h = jnp.maximum(h, 0.0)
    h = jnp.dot(h, w3[...], preferred_element_type=jnp.float32) + b3[...]
    h = jnp.maximum(h, 0.0)
    o = jnp.dot(h, wout[...], preferred_element_type=jnp.float32)
    o_ref[...] = o[:, 0] + jnp.sum(r_ref[...], axis=1)


_BLK = 512


def _mlp_call(s, r, w1, b1, w2, b2, w3, b3, wout):
    full = lambda i: (0, 0)
    return pl.pallas_call(
        _mlp_body,
        grid=(_B // _BLK,),
        in_specs=[
            pl.BlockSpec((_BLK, _F * _D), lambda i: (i, 0)),
            pl.BlockSpec((_BLK, 16), lambda i: (i, 0)),
            pl.BlockSpec((_F * _D, 400), full),
            pl.BlockSpec((400,), lambda i: (0,)),
            pl.BlockSpec((400, 400), full),
            pl.BlockSpec((400,), lambda i: (0,)),
            pl.BlockSpec((400, 400), full),
            pl.BlockSpec((400,), lambda i: (0,)),
            pl.BlockSpec((400, 1), full),
        ],
        out_specs=pl.BlockSpec((_BLK,), lambda i: (i,)),
        out_shape=jax.ShapeDtypeStruct((_B,), jnp.float32),
    )(s, r, w1, b1, w2, b2, w3, b3, wout)


def kernel(x, fc_w, bias, emb, W1, b1, g1, be1, W2, b2, g2, be2, W3, b3, g3, be3, Wout, bout):
    offs = jnp.asarray(_OFFS)
    xo = x + offs[None, :]
    xo_flat = xo.reshape(-1)

    fctab = jnp.broadcast_to(fc_w, (_V, _D))
    s, r = _sc_call(emb, fctab, xo_flat)

    inv = 1.0 / math.sqrt(1.0 + 1e-5)
    s1 = g1 * inv
    s2 = g2 * inv
    s3 = g3 * inv
    w1 = W1 * s1[None, :]
    b1f = b1 * s1 + be1
    w2 = W2 * s2[None, :]
    b2f = b2 * s2 + be2
    w3 = W3 * s3[None, :]
    b3f = b3 * s3 + be3

    out = _mlp_call(s, r, w1, b1f, w2, b2f, w3, b3f, Wout)
    return out + (bias[0] + bout[0])


# speedup vs baseline: 1.1618x; 1.0406x over previous
import math

import jax
import jax.numpy as jnp
import numpy as np
from jax import lax
from jax.experimental import pallas as pl
from jax.experimental.pallas import tpu as pltpu
from jax.experimental.pallas import tpu_sc as plsc

_FIELD_DIMS = [3846] * 25 + [3850]
_V = sum(_FIELD_DIMS)
_F = 26
_D = 16
_B = 4096
_OFFS = np.concatenate([[0], np.cumsum(_FIELD_DIMS)[:-1]]).astype(np.int32)

_NC = 2
_NS = 16
_NW = _NC * _NS
_RPW = _B // _NW
_C = 4
_CH = _RPW // _C
_IDXN = _C * _F


def _sc_body(emb, fctab, xo, s_out, r_out, idx_v, slab, fcbuf, sbuf, rbuf, sem):
    wid = lax.axis_index("s") * _NC + lax.axis_index("c")
    row0 = wid * _RPW

    def chunk_body(g, carry):
        start = (row0 + g * _C) * _F
        pltpu.sync_copy(xo.at[pl.ds(start, _IDXN)], idx_v)
        descs = []
        for i in range(_F):
            descs.append(
                pltpu.async_copy(
                    emb.at[i].at[idx_v], slab.at[pl.ds(i * _IDXN, _IDXN)], sem
                )
            )
        descs.append(pltpu.async_copy(fctab.at[idx_v], fcbuf, sem))
        for d in descs:
            d.wait()

        def row_body(c, carry2):
            base = c * _F
            s_cols = [slab[j * _IDXN + base + j, :] for j in range(_F)]
            ffm = jnp.zeros((16,), jnp.float32)
            lin = jnp.zeros((16,), jnp.float32)
            for jj in range(1, _F):
                for ii in range(jj):
                    u = slab[ii * _IDXN + base + jj, :]
                    v = slab[jj * _IDXN + base + ii, :]
                    ffm = ffm + u * v
                    s_cols[jj] = s_cols[jj] + u
                    s_cols[ii] = s_cols[ii] + v
            for j in range(_F):
                lin = lin + fcbuf[base + j, :]
            rbuf[c, :] = ffm + lin * (1.0 / 16.0)
            for j in range(_F):
                sbuf[c, pl.ds(j * 16, 16)] = s_cols[j]
            return carry2

        lax.fori_loop(0, _C, row_body, 0)
        pltpu.sync_copy(sbuf, s_out.at[pl.ds(row0 + g * _C, _C)])
        pltpu.sync_copy(rbuf, r_out.at[pl.ds(row0 + g * _C, _C)])
        return carry

    lax.fori_loop(0, _CH, chunk_body, 0)


_sc_call = pl.kernel(
    _sc_body,
    out_type=[
        jax.ShapeDtypeStruct((_B, _F * _D), jnp.float32),
        jax.ShapeDtypeStruct((_B, 16), jnp.float32),
    ],
    mesh=plsc.VectorSubcoreMesh(
        core_axis_name="c", subcore_axis_name="s", num_cores=_NC, num_subcores=_NS
    ),
    scratch_types=[
        pltpu.VMEM((_IDXN,), jnp.int32),
        pltpu.VMEM((_F * _IDXN, _D), jnp.float32),
        pltpu.VMEM((_IDXN, 16), jnp.float32),
        pltpu.VMEM((_C, _F * _D), jnp.float32),
        pltpu.VMEM((_C, 16), jnp.float32),
        pltpu.SemaphoreType.DMA,
    ],
    compiler_params=pltpu.CompilerParams(use_tc_tiling_on_sc=False),
)


def _mlp_body(s_ref, r_ref, w1, b1, w2, b2, w3, b3, wout, o_ref):
    h = jnp.dot(s_ref[...], w1[...], preferred_element_type=jnp.float32) + b1[...]
    h = jnp.maximum(h, 0.0)
    h = jnp.dot(h, w2[...], preferred_element_type=jnp.float32) + b2[...]
    h = jnp.maximum(h, 0.0)
    h = jnp.dot(h, w3[...], preferred_element_type=jnp.float32) + b3[...]
    h = jnp.maximum(h, 0.0)
    o = jnp.dot(h, wout[...], preferred_element_type=jnp.float32)
    o_ref[...] = o[:, 0] + jnp.sum(r_ref[...], axis=1)


_BLK = 512


def _mlp_call(s, r, w1, b1, w2, b2, w3, b3, wout):
    full = lambda i: (0, 0)
    return pl.pallas_call(
        _mlp_body,
        grid=(_B // _BLK,),
        in_specs=[
            pl.BlockSpec((_BLK, _F * _D), lambda i: (i, 0)),
            pl.BlockSpec((_BLK, 16), lambda i: (i, 0)),
            pl.BlockSpec((_F * _D, 400), full),
            pl.BlockSpec((400,), lambda i: (0,)),
            pl.BlockSpec((400, 400), full),
            pl.BlockSpec((400,), lambda i: (0,)),
            pl.BlockSpec((400, 400), full),
            pl.BlockSpec((400,), lambda i: (0,)),
            pl.BlockSpec((400, 1), full),
        ],
        out_specs=pl.BlockSpec((_BLK,), lambda i: (i,)),
        out_shape=jax.ShapeDtypeStruct((_B,), jnp.float32),
    )(s, r, w1, b1, w2, b2, w3, b3, wout)


def kernel(x, fc_w, bias, emb, W1, b1, g1, be1, W2, b2, g2, be2, W3, b3, g3, be3, Wout, bout):
    offs = jnp.asarray(_OFFS)
    xo_flat = (x + offs[None, :]).reshape(-1)
    fctab = jnp.broadcast_to(fc_w, (_V, _D))

    s, r = _sc_call(emb, fctab, xo_flat)

    inv = 1.0 / math.sqrt(1.0 + 1e-5)
    s1 = g1 * inv
    s2 = g2 * inv
    s3 = g3 * inv
    w1 = W1 * s1[None, :]
    b1f = b1 * s1 + be1
    w2 = W2 * s2[None, :]
    b2f = b2 * s2 + be2
    w3 = W3 * s3[None, :]
    b3f = b3 * s3 + be3

    out = _mlp_call(s, r, w1, b1f, w2, b2f, w3, b3f, Wout)
    return out + (bias[0] + bout[0])
